# Initial kernel scaffold; baseline (speedup 1.0000x reference)
import jax
import jax.numpy as jnp
from jax.experimental import pallas as pl


def kernel(x, batch, W, b, gamma, beta):
    raise NotImplementedError("write your pallas kernel here")


# speedup vs baseline: 4.5054x; 4.5054x over previous
import functools

import jax
import jax.numpy as jnp
from jax import lax
from jax.experimental import pallas as pl
from jax.experimental.pallas import tpu as pltpu
from jax.experimental.pallas import tpu_sc as plsc

N = 10000
D = 128
K = 20
NB = 8
EPS = 1e-5

NPAD = 10240
TR = 256
TN = 128
EPD = NPAD * K

BIG1 = 1e30
BIG2 = 2e30


def _knn_pq_body(xr_ref, xall_ref, btr_ref, btc_ref, sqr_ref, sqc_ref,
                 wat_ref, wdt_ref, b_ref,
                 idx_ref, p_ref, q_ref, dist_ref):
    i = pl.program_id(0)
    xr = xr_ref[...]
    xall = xall_ref[...]

    p_ref[...] = jnp.dot(xr, wat_ref[...],
                         preferred_element_type=jnp.float32) + b_ref[...]
    q_ref[...] = jnp.dot(xr, wdt_ref[...],
                         preferred_element_type=jnp.float32)

    main = lax.dot_general(xr, xall, (((1,), (1,)), ((), ())),
                           preferred_element_type=jnp.float32)
    dist = (sqr_ref[...] + sqc_ref[...]) - 2.0 * main

    colio = lax.broadcasted_iota(jnp.int32, (TR, NPAD), 1)
    rowio = lax.broadcasted_iota(jnp.int32, (TR, NPAD), 0) + i * TR
    valid = (btr_ref[...] == btc_ref[...]) & (colio != rowio)
    dist_ref[...] = jnp.where(valid, dist, BIG1)

    kio = lax.broadcasted_iota(jnp.int32, (TR, K), 1)

    def body(k, idxacc):
        d = dist_ref[...]
        m = jnp.min(d, axis=1, keepdims=True)
        j = jnp.min(jnp.where(d == m, colio, NPAD), axis=1,
                    keepdims=True)
        dist_ref[...] = jnp.where(colio == j, BIG2, d)
        return jnp.where(kio == k, j, idxacc)

    idx_ref[...] = lax.fori_loop(0, K, body,
                                 jnp.zeros((TR, K), jnp.int32))


def _knn_pq(xp, bt_row, bt_col, sq_row, sq_col, wat, wdt, bvec):
    nt = NPAD // TR
    return pl.pallas_call(
        _knn_pq_body,
        grid=(nt,),
        in_specs=[
            pl.BlockSpec((TR, D), lambda i: (i, 0)),
            pl.BlockSpec((NPAD, D), lambda i: (0, 0)),
            pl.BlockSpec((TR, 1), lambda i: (i, 0)),
            pl.BlockSpec((1, NPAD), lambda i: (0, 0)),
            pl.BlockSpec((TR, 1), lambda i: (i, 0)),
            pl.BlockSpec((1, NPAD), lambda i: (0, 0)),
            pl.BlockSpec((D, D), lambda i: (0, 0)),
            pl.BlockSpec((D, D), lambda i: (0, 0)),
            pl.BlockSpec((1, D), lambda i: (0, 0)),
        ],
        out_specs=[
            pl.BlockSpec((TR, K), lambda i: (i, 0)),
            pl.BlockSpec((TR, D), lambda i: (i, 0)),
            pl.BlockSpec((TR, D), lambda i: (i, 0)),
        ],
        out_shape=[
            jax.ShapeDtypeStruct((NPAD, K), jnp.int32),
            jax.ShapeDtypeStruct((NPAD, D), jnp.float32),
            jax.ShapeDtypeStruct((NPAD, D), jnp.float32),
        ],
        scratch_shapes=[pltpu.VMEM((TR, NPAD), jnp.float32)],
    )(xp, xp, bt_row, bt_col, sq_row, sq_col, wat, wdt, bvec)


_SC_NC = 2
_SC_NS = 16
_SC_NW = _SC_NC * _SC_NS
_SC_BPW = EPD // _SC_NW
_SC_CH = 640
_SC_NCH = _SC_BPW // _SC_CH


def _sc_gather(qp, idx_flat):
    mesh = plsc.VectorSubcoreMesh(core_axis_name="c", subcore_axis_name="s")

    @functools.partial(
        pl.kernel,
        mesh=mesh,
        out_type=jax.ShapeDtypeStruct((EPD, D), jnp.float32),
        scratch_types=[
            pltpu.VMEM((_SC_CH,), jnp.int32),
            pltpu.VMEM((_SC_CH, D), jnp.float32),
            pltpu.SemaphoreType.DMA,
        ],
    )
    def gather_k(q_hbm, idx_hbm, out_hbm, idx_v, rows_v, sem):
        wid = lax.axis_index("s") * _SC_NC + lax.axis_index("c")
        base = wid * _SC_BPW
        for t in range(_SC_NCH):
            off = base + t * _SC_CH
            pltpu.sync_copy(idx_hbm.at[pl.ds(off, _SC_CH)], idx_v)
            pltpu.async_copy(q_hbm.at[idx_v], rows_v, sem).wait()
            pltpu.sync_copy(rows_v, out_hbm.at[pl.ds(off, _SC_CH)])

    return gather_k(qp, idx_flat)


def _stats_body(p_ref, g_ref, s1_ref, s2_ref):
    i = pl.program_id(0)

    @pl.when(i == 0)
    def _():
        s1_ref[...] = jnp.zeros_like(s1_ref)
        s2_ref[...] = jnp.zeros_like(s2_ref)

    rowid = lax.broadcasted_iota(jnp.int32, (TN, 1), 0) + i * TN
    real = (rowid < N)[:, :, None]
    h = p_ref[...][:, None, :] + g_ref[...]
    hm = jnp.where(real, h, 0.0)
    s1_ref[...] += jnp.sum(jnp.sum(hm, axis=1), axis=0, keepdims=True)
    s2_ref[...] += jnp.sum(jnp.sum(hm * hm, axis=1), axis=0, keepdims=True)


def _stats(pp, g3):
    nt = NPAD // TN
    return pl.pallas_call(
        _stats_body,
        grid=(nt,),
        in_specs=[
            pl.BlockSpec((TN, D), lambda i: (i, 0)),
            pl.BlockSpec((TN, K, D), lambda i: (i, 0, 0)),
        ],
        out_specs=[
            pl.BlockSpec((1, D), lambda i: (0, 0)),
            pl.BlockSpec((1, D), lambda i: (0, 0)),
        ],
        out_shape=[
            jax.ShapeDtypeStruct((1, D), jnp.float32),
            jax.ShapeDtypeStruct((1, D), jnp.float32),
        ],
    )(pp, g3)


def _out_body(p_ref, g_ref, a_ref, c_ref, o_ref):
    h = p_ref[...][:, None, :] + g_ref[...]
    y = jnp.maximum(a_ref[...][None] * h + c_ref[...][None], 0.0)
    o_ref[...] = jnp.sum(y, axis=1)


def _out(pp, g3, avec, cvec):
    nt = NPAD // TN
    return pl.pallas_call(
        _out_body,
        grid=(nt,),
        in_specs=[
            pl.BlockSpec((TN, D), lambda i: (i, 0)),
            pl.BlockSpec((TN, K, D), lambda i: (i, 0, 0)),
            pl.BlockSpec((1, D), lambda i: (0, 0)),
            pl.BlockSpec((1, D), lambda i: (0, 0)),
        ],
        out_specs=pl.BlockSpec((TN, D), lambda i: (i, 0)),
        out_shape=jax.ShapeDtypeStruct((NPAD, D), jnp.float32),
    )(pp, g3, avec, cvec)


@jax.jit
def kernel(x, batch, W, b, gamma, beta):
    xp = jnp.zeros((NPAD, D), jnp.float32).at[:N].set(x)
    bp = jnp.concatenate(
        [batch.astype(jnp.int32),
         jnp.full((NPAD - N,), NB, jnp.int32)])
    bt_row = bp[:, None]
    bt_col = bp[None, :]
    sq = jnp.sum(xp * xp, axis=1)
    sq_row = sq[:, None]
    sq_col = sq[None, :]

    wa = W[:, :D]
    wd = W[:, D:] - wa
    wat = wa.T
    wdt = wd.T
    bvec = b[None, :]

    idx, pp, qp = _knn_pq(xp, bt_row, bt_col, sq_row, sq_col,
                          wat, wdt, bvec)

    g = _sc_gather(qp, idx.reshape(-1))
    g3 = g.reshape(NPAD, K, D)

    s1, s2 = _stats(pp, g3)

    e = jnp.float32(N * K)
    mu = s1 / e
    var = s2 / e - mu * mu
    inv = lax.rsqrt(var + EPS)
    avec = gamma[None, :] * inv
    cvec = beta[None, :] - mu * avec

    out = _out(pp, g3, avec, cvec)
    return out[:N]


# speedup vs baseline: 11.6148x; 2.5780x over previous
import functools

import jax
import jax.numpy as jnp
from jax import lax
from jax.experimental import pallas as pl
from jax.experimental.pallas import tpu as pltpu
from jax.experimental.pallas import tpu_sc as plsc

N = 10000
D = 128
K = 20
NB = 8
EPS = 1e-5

NPAD = 10240
TR = 256
TN = 128
EPD = NPAD * K

BIG1 = 1e30
BIG2 = 2e30

WIN = 3072


def _knn_pq_body(sinfo_ref, xr_ref, xall_ref, btr_ref, btc_ref,
                 sqr_ref, sqc_ref, wat_ref, wdt_ref, b_ref,
                 idx_ref, p_ref, q_ref, dist_ref):
    i = pl.program_id(0)
    xr = xr_ref[...]

    p_ref[...] = jnp.dot(xr, wat_ref[...],
                         preferred_element_type=jnp.float32) + b_ref[...]
    q_ref[...] = jnp.dot(xr, wdt_ref[...],
                         preferred_element_type=jnp.float32)

    sq_r = sqr_ref[...]
    btr = btr_ref[...]
    a = sinfo_ref[0, i] * 128
    e = sinfo_ref[1, i]
    fast = (e - a) <= WIN
    kio = lax.broadcasted_iota(jnp.int32, (TR, K), 1)

    def build_extract(w, base):
        xw = xall_ref[pl.ds(base, w), :]
        main = lax.dot_general(xr, xw, (((1,), (1,)), ((), ())),
                               preferred_element_type=jnp.float32)
        dist = (sq_r + sqc_ref[:, pl.ds(base, w)]) - 2.0 * main
        colio = lax.broadcasted_iota(jnp.int32, (TR, w), 1)
        gcol = colio + base
        growio = lax.broadcasted_iota(jnp.int32, (TR, w), 0) + i * TR
        valid = (btr == btc_ref[:, pl.ds(base, w)]) & (gcol != growio)
        dist_ref[:, :w] = jnp.where(valid, dist, BIG1)

        def body(k, idxacc):
            d = dist_ref[:, :w]
            m = jnp.min(d, axis=1, keepdims=True)
            j = jnp.min(jnp.where(d == m, colio, w), axis=1,
                        keepdims=True)
            dist_ref[:, :w] = jnp.where(colio == j, BIG2, d)
            return jnp.where(kio == k, j + base, idxacc)

        idx_ref[...] = lax.fori_loop(0, K, body,
                                     jnp.zeros((TR, K), jnp.int32))

    @pl.when(fast)
    def _():
        build_extract(WIN, a)

    @pl.when(jnp.logical_not(fast))
    def _():
        build_extract(NPAD, 0)


def _knn_pq(sinfo, xp, bt_row, bt_col, sq_row, sq_col, wat, wdt, bvec):
    nt = NPAD // TR
    grid_spec = pltpu.PrefetchScalarGridSpec(
        num_scalar_prefetch=1,
        grid=(nt,),
        in_specs=[
            pl.BlockSpec((TR, D), lambda i, s: (i, 0)),
            pl.BlockSpec((NPAD, D), lambda i, s: (0, 0)),
            pl.BlockSpec((TR, 1), lambda i, s: (i, 0)),
            pl.BlockSpec((1, NPAD), lambda i, s: (0, 0)),
            pl.BlockSpec((TR, 1), lambda i, s: (i, 0)),
            pl.BlockSpec((1, NPAD), lambda i, s: (0, 0)),
            pl.BlockSpec((D, D), lambda i, s: (0, 0)),
            pl.BlockSpec((D, D), lambda i, s: (0, 0)),
            pl.BlockSpec((1, D), lambda i, s: (0, 0)),
        ],
        out_specs=[
            pl.BlockSpec((TR, K), lambda i, s: (i, 0)),
            pl.BlockSpec((TR, D), lambda i, s: (i, 0)),
            pl.BlockSpec((TR, D), lambda i, s: (i, 0)),
        ],
        scratch_shapes=[pltpu.VMEM((TR, NPAD), jnp.float32)],
    )
    return pl.pallas_call(
        _knn_pq_body,
        grid_spec=grid_spec,
        out_shape=[
            jax.ShapeDtypeStruct((NPAD, K), jnp.int32),
            jax.ShapeDtypeStruct((NPAD, D), jnp.float32),
            jax.ShapeDtypeStruct((NPAD, D), jnp.float32),
        ],
    )(sinfo, xp, xp, bt_row, bt_col, sq_row, sq_col, wat, wdt, bvec)


_SC_NC = 2
_SC_NS = 16
_SC_NW = _SC_NC * _SC_NS
_SC_BPW = EPD // _SC_NW
_SC_CH = 640
_SC_NCH = _SC_BPW // _SC_CH


def _sc_gather(qp, idx_flat):
    mesh = plsc.VectorSubcoreMesh(core_axis_name="c", subcore_axis_name="s")

    @functools.partial(
        pl.kernel,
        mesh=mesh,
        out_type=jax.ShapeDtypeStruct((EPD, D), jnp.float32),
        scratch_types=[
            pltpu.VMEM((_SC_CH,), jnp.int32),
            pltpu.VMEM((_SC_CH, D), jnp.float32),
            pltpu.SemaphoreType.DMA,
        ],
    )
    def gather_k(q_hbm, idx_hbm, out_hbm, idx_v, rows_v, sem):
        wid = lax.axis_index("s") * _SC_NC + lax.axis_index("c")
        base = wid * _SC_BPW
        for t in range(_SC_NCH):
            off = base + t * _SC_CH
            pltpu.sync_copy(idx_hbm.at[pl.ds(off, _SC_CH)], idx_v)
            pltpu.async_copy(q_hbm.at[idx_v], rows_v, sem).wait()
            pltpu.sync_copy(rows_v, out_hbm.at[pl.ds(off, _SC_CH)])

    return gather_k(qp, idx_flat)


def _stats_body(p_ref, g_ref, s1_ref, s2_ref):
    i = pl.program_id(0)

    @pl.when(i == 0)
    def _():
        s1_ref[...] = jnp.zeros_like(s1_ref)
        s2_ref[...] = jnp.zeros_like(s2_ref)

    rowid = lax.broadcasted_iota(jnp.int32, (TN, 1), 0) + i * TN
    real = (rowid < N)[:, :, None]
    h = p_ref[...][:, None, :] + g_ref[...]
    hm = jnp.where(real, h, 0.0)
    s1_ref[...] += jnp.sum(jnp.sum(hm, axis=1), axis=0, keepdims=True)
    s2_ref[...] += jnp.sum(jnp.sum(hm * hm, axis=1), axis=0, keepdims=True)


def _stats(pp, g3):
    nt = NPAD // TN
    return pl.pallas_call(
        _stats_body,
        grid=(nt,),
        in_specs=[
            pl.BlockSpec((TN, D), lambda i: (i, 0)),
            pl.BlockSpec((TN, K, D), lambda i: (i, 0, 0)),
        ],
        out_specs=[
            pl.BlockSpec((1, D), lambda i: (0, 0)),
            pl.BlockSpec((1, D), lambda i: (0, 0)),
        ],
        out_shape=[
            jax.ShapeDtypeStruct((1, D), jnp.float32),
            jax.ShapeDtypeStruct((1, D), jnp.float32),
        ],
    )(pp, g3)


def _out_body(p_ref, g_ref, a_ref, c_ref, o_ref):
    h = p_ref[...][:, None, :] + g_ref[...]
    y = jnp.maximum(a_ref[...][None] * h + c_ref[...][None], 0.0)
    o_ref[...] = jnp.sum(y, axis=1)


def _out(pp, g3, avec, cvec):
    nt = NPAD // TN
    return pl.pallas_call(
        _out_body,
        grid=(nt,),
        in_specs=[
            pl.BlockSpec((TN, D), lambda i: (i, 0)),
            pl.BlockSpec((TN, K, D), lambda i: (i, 0, 0)),
            pl.BlockSpec((1, D), lambda i: (0, 0)),
            pl.BlockSpec((1, D), lambda i: (0, 0)),
        ],
        out_specs=pl.BlockSpec((TN, D), lambda i: (i, 0)),
        out_shape=jax.ShapeDtypeStruct((NPAD, D), jnp.float32),
    )(pp, g3, avec, cvec)


@jax.jit
def kernel(x, batch, W, b, gamma, beta):
    xp = jnp.zeros((NPAD, D), jnp.float32).at[:N].set(x)
    bp = jnp.concatenate(
        [batch.astype(jnp.int32),
         jnp.full((NPAD - N,), NB, jnp.int32)])
    bt_row = bp[:, None]
    bt_col = bp[None, :]
    sq = jnp.sum(xp * xp, axis=1)
    sq_row = sq[:, None]
    sq_col = sq[None, :]

    heads = bp[::TR]
    tails = bp[TR - 1::TR]
    starts = jnp.searchsorted(bp, heads, side="left").astype(jnp.int32)
    ends = jnp.searchsorted(bp, tails, side="right").astype(jnp.int32)
    astarts = jnp.minimum(starts // 128, (NPAD - WIN) // 128)
    sinfo = jnp.stack([astarts, ends])

    wa = W[:, :D]
    wd = W[:, D:] - wa
    wat = wa.T
    wdt = wd.T
    bvec = b[None, :]

    idx, pp, qp = _knn_pq(sinfo, xp, bt_row, bt_col, sq_row, sq_col,
                          wat, wdt, bvec)

    g = _sc_gather(qp, idx.reshape(-1))
    g3 = g.reshape(NPAD, K, D)

    s1, s2 = _stats(pp, g3)

    e = jnp.float32(N * K)
    mu = s1 / e
    var = s2 / e - mu * mu
    inv = lax.rsqrt(var + EPS)
    avec = gamma[None, :] * inv
    cvec = beta[None, :] - mu * avec

    out = _out(pp, g3, avec, cvec)
    return out[:N]
